# baseline (speedup 1.0000x reference)
import functools

import jax
import jax.numpy as jnp
from jax import lax
from jax.experimental import pallas as pl
from jax.experimental.pallas import tpu as pltpu
from jax.experimental.pallas import tpu_sc as plsc


def kernel(x, pe):
    b, t, d = x.shape
    nc, ns, nl = 2, 16, 16
    nw = nc * ns
    rows_per_w = t // nw
    chunk = 16
    n_chunks = rows_per_w // chunk

    mesh = plsc.VectorSubcoreMesh(
        core_axis_name="c", subcore_axis_name="s", num_cores=nc, num_subcores=ns
    )

    @functools.partial(
        pl.kernel,
        out_type=jax.ShapeDtypeStruct((b, t, d), jnp.float32),
        mesh=mesh,
        scratch_types=[
            pltpu.VMEM((rows_per_w, d), jnp.float32),
            pltpu.VMEM((chunk, d), jnp.float32),
        ],
    )
    def sc_fn(x_hbm, pe_hbm, out_hbm, pe_v, buf_v):
        wid = lax.axis_index("s") * nc + lax.axis_index("c")
        t0 = wid * rows_per_w
        pltpu.sync_copy(pe_hbm.at[pl.ds(t0, rows_per_w)], pe_v)

        def outer_body(i, carry):
            bb = i // n_chunks
            c = i % n_chunks
            row0 = t0 + c * chunk
            pltpu.sync_copy(x_hbm.at[bb, pl.ds(row0, chunk)], buf_v)

            def row_body(r, cr):
                for j in range(d // nl):
                    sl = pl.ds(j * nl, nl)
                    plsc.addupdate(buf_v.at[r, sl], pe_v[c * chunk + r, sl])
                return cr

            lax.fori_loop(0, chunk, row_body, 0)
            pltpu.sync_copy(buf_v, out_hbm.at[bb, pl.ds(row0, chunk)])
            return carry

        lax.fori_loop(0, b * n_chunks, outer_body, 0)

    return sc_fn(x, pe)


# speedup vs baseline: 1.2467x; 1.2467x over previous
import functools

import jax
import jax.numpy as jnp
from jax import lax
from jax.experimental import pallas as pl
from jax.experimental.pallas import tpu as pltpu
from jax.experimental.pallas import tpu_sc as plsc


def kernel(x, pe):
    b, t, d = x.shape
    nc, ns, nl = 2, 16, 16
    nw = nc * ns
    rows_per_w = t // nw
    chunk = 8
    nbuf = 4
    n_steps = b * (rows_per_w // chunk)
    n_chunks = rows_per_w // chunk

    mesh = plsc.VectorSubcoreMesh(
        core_axis_name="c", subcore_axis_name="s", num_cores=nc, num_subcores=ns
    )

    @functools.partial(
        pl.kernel,
        out_type=jax.ShapeDtypeStruct((b, t, d), jnp.float32),
        mesh=mesh,
        scratch_types=[
            pltpu.VMEM((rows_per_w, d), jnp.float32),
            pltpu.VMEM((nbuf, chunk, d), jnp.float32),
            pltpu.SemaphoreType.DMA((nbuf,)),
            pltpu.SemaphoreType.DMA((nbuf,)),
        ],
    )
    def sc_fn(x_hbm, pe_hbm, out_hbm, pe_v, xb, in_sems, out_sems):
        wid = lax.axis_index("s") * nc + lax.axis_index("c")
        t0 = wid * rows_per_w
        pltpu.sync_copy(pe_hbm.at[pl.ds(t0, rows_per_w)], pe_v)

        def issue_in(i, q):
            bb = i // n_chunks
            row0 = t0 + (i % n_chunks) * chunk
            pltpu.async_copy(
                x_hbm.at[bb, pl.ds(row0, chunk)], xb.at[q], in_sems.at[q]
            )

        def wait_in(q):
            pltpu.make_async_copy(
                x_hbm.at[0, pl.ds(0, chunk)], xb.at[q], in_sems.at[q]
            ).wait()

        def issue_out(i, q):
            bb = i // n_chunks
            row0 = t0 + (i % n_chunks) * chunk
            pltpu.async_copy(
                xb.at[q], out_hbm.at[bb, pl.ds(row0, chunk)], out_sems.at[q]
            )

        def wait_out(q):
            pltpu.make_async_copy(
                xb.at[q], out_hbm.at[0, pl.ds(0, chunk)], out_sems.at[q]
            ).wait()

        for k in range(nbuf - 1):
            issue_in(k, k)

        def step_body(s, carry):
            for q in range(nbuf):
                i = s * nbuf + q
                c = i % n_chunks
                wait_in(q)

                def row_body(r, cr):
                    for j in range(d // nl):
                        sl = pl.ds(j * nl, nl)
                        plsc.addupdate(xb.at[q, r, sl], pe_v[c * chunk + r, sl])
                    return cr

                lax.fori_loop(0, chunk, row_body, 0)
                issue_out(i, q)
                nxt = i + nbuf - 1
                qn = (q + nbuf - 1) % nbuf

                @pl.when(nxt < n_steps)
                def _prefetch():
                    @pl.when(nxt >= nbuf)
                    def _drain():
                        wait_out(qn)

                    issue_in(nxt, qn)

            return carry

        lax.fori_loop(0, n_steps // nbuf, step_body, 0)
        for q in range(nbuf):
            wait_out(q)

    return sc_fn(x, pe)


# speedup vs baseline: 1.5120x; 1.2128x over previous
import functools

import jax
import jax.numpy as jnp
from jax import lax
from jax.experimental import pallas as pl
from jax.experimental.pallas import tpu as pltpu
from jax.experimental.pallas import tpu_sc as plsc


def kernel(x, pe):
    b, t, d = x.shape
    nc, ns, nl = 2, 16, 16
    nw = nc * ns
    rows_per_w = t // nw
    chunk = 8
    nbuf = 4
    n_steps = b * (rows_per_w // chunk)
    n_chunks = rows_per_w // chunk

    mesh = plsc.VectorSubcoreMesh(
        core_axis_name="c", subcore_axis_name="s", num_cores=nc, num_subcores=ns
    )

    @functools.partial(
        pl.kernel,
        out_type=jax.ShapeDtypeStruct((b, t, d), jnp.float32),
        mesh=mesh,
        scratch_types=[
            pltpu.VMEM((rows_per_w, d), jnp.float32),
            pltpu.VMEM((nbuf, chunk, d), jnp.float32),
            pltpu.SemaphoreType.DMA((nbuf,)),
            pltpu.SemaphoreType.DMA((nbuf,)),
        ],
    )
    def sc_fn(x_hbm, pe_hbm, out_hbm, pe_v, xb, in_sems, out_sems):
        wid = lax.axis_index("s") * nc + lax.axis_index("c")
        t0 = wid * rows_per_w
        pltpu.sync_copy(pe_hbm.at[pl.ds(t0, rows_per_w)], pe_v)

        def issue_in(i, q):
            bb = i // n_chunks
            row0 = t0 + (i % n_chunks) * chunk
            pltpu.async_copy(
                x_hbm.at[bb, pl.ds(row0, chunk)], xb.at[q], in_sems.at[q]
            )

        def wait_in(q):
            pltpu.make_async_copy(
                x_hbm.at[0, pl.ds(0, chunk)], xb.at[q], in_sems.at[q]
            ).wait()

        def issue_out(i, q):
            bb = i // n_chunks
            row0 = t0 + (i % n_chunks) * chunk
            pltpu.async_copy(
                xb.at[q], out_hbm.at[bb, pl.ds(row0, chunk)], out_sems.at[q]
            )

        def wait_out(q):
            pltpu.make_async_copy(
                xb.at[q], out_hbm.at[0, pl.ds(0, chunk)], out_sems.at[q]
            ).wait()

        for k in range(nbuf - 1):
            issue_in(k, k)

        def step_body(s, carry):
            for q in range(nbuf):
                i = s * nbuf + q
                c = i % n_chunks
                wait_in(q)

                @plsc.parallel_loop(0, chunk, 1, unroll=2)
                def row_body(r):
                    for j in range(d // nl):
                        sl = pl.ds(j * nl, nl)
                        plsc.addupdate(xb.at[q, r, sl], pe_v[c * chunk + r, sl])
                issue_out(i, q)
                nxt = i + nbuf - 1
                qn = (q + nbuf - 1) % nbuf

                @pl.when(nxt < n_steps)
                def _prefetch():
                    @pl.when(nxt >= nbuf)
                    def _drain():
                        wait_out(qn)

                    issue_in(nxt, qn)

            return carry

        lax.fori_loop(0, n_steps // nbuf, step_body, 0)
        for q in range(nbuf):
            wait_out(q)

    return sc_fn(x, pe)


# speedup vs baseline: 2.2547x; 1.4913x over previous
import functools

import jax
import jax.numpy as jnp
from jax import lax
from jax.experimental import pallas as pl
from jax.experimental.pallas import tpu as pltpu
from jax.experimental.pallas import tpu_sc as plsc


def kernel(x, pe):
    b, t, d = x.shape
    nc, ns, nl = 2, 16, 16
    nw = nc * ns
    rows_per_w = t // nw
    chunk = 8
    n_sets = rows_per_w // chunk

    mesh = plsc.VectorSubcoreMesh(
        core_axis_name="c", subcore_axis_name="s", num_cores=nc, num_subcores=ns
    )

    @functools.partial(
        pl.kernel,
        out_type=jax.ShapeDtypeStruct((b, t, d), jnp.float32),
        mesh=mesh,
        scratch_types=[
            pltpu.VMEM((2, chunk, d), jnp.float32),
            pltpu.VMEM((2, b, chunk, d), jnp.float32),
            pltpu.SemaphoreType.DMA((2,)),
            pltpu.SemaphoreType.DMA((2,)),
        ],
    )
    def sc_fn(x_hbm, pe_hbm, out_hbm, pe2, xb, in_sems, out_sems):
        wid = lax.axis_index("s") * nc + lax.axis_index("c")
        t0 = wid * rows_per_w

        def issue_in(s, p):
            row0 = t0 + s * chunk
            pltpu.async_copy(pe_hbm.at[pl.ds(row0, chunk)], pe2.at[p], in_sems.at[p])
            for bb in range(b):
                pltpu.async_copy(
                    x_hbm.at[bb, pl.ds(row0, chunk)], xb.at[p, bb], in_sems.at[p]
                )

        def wait_in(p):
            pltpu.make_async_copy(
                pe_hbm.at[pl.ds(0, chunk)], pe2.at[p], in_sems.at[p]
            ).wait()
            for bb in range(b):
                pltpu.make_async_copy(
                    x_hbm.at[0, pl.ds(0, chunk)], xb.at[p, bb], in_sems.at[p]
                ).wait()

        def issue_out(s, p):
            row0 = t0 + s * chunk
            for bb in range(b):
                pltpu.async_copy(
                    xb.at[p, bb], out_hbm.at[bb, pl.ds(row0, chunk)], out_sems.at[p]
                )

        def wait_out(p):
            for bb in range(b):
                pltpu.make_async_copy(
                    xb.at[p, bb], out_hbm.at[0, pl.ds(0, chunk)], out_sems.at[p]
                ).wait()

        issue_in(0, 0)

        def set_body(h, carry):
            for p in range(2):
                s = h * 2 + p
                wait_in(p)

                @pl.when(s + 1 < n_sets)
                def _prefetch():
                    @pl.when(s >= 1)
                    def _drain():
                        wait_out(1 - p)

                    issue_in(s + 1, 1 - p)

                @plsc.parallel_loop(0, chunk, 1, unroll=2)
                def row_body(r):
                    grp = 4
                    for g in range(0, d // nl, grp):
                        sls = [pl.ds((g + u) * nl, nl) for u in range(grp)]
                        vals = [pe2[p, r, sls[u]] for u in range(grp)]
                        for bb in range(b):
                            for u in range(grp):
                                xb[p, bb, r, sls[u]] = xb[p, bb, r, sls[u]] + vals[u]

                issue_out(s, p)

            return carry

        lax.fori_loop(0, n_sets // 2, set_body, 0)
        for p in range(2):
            wait_out(p)

    return sc_fn(x, pe)


# speedup vs baseline: 2.2861x; 1.0139x over previous
import functools

import jax
import jax.numpy as jnp
from jax import lax
from jax.experimental import pallas as pl
from jax.experimental.pallas import tpu as pltpu
from jax.experimental.pallas import tpu_sc as plsc


def kernel(x, pe):
    b, t, d = x.shape
    nc, ns, nl = 2, 16, 16
    nw = nc * ns
    rows_per_w = t // nw
    chunk = 8
    n_sets = rows_per_w // chunk

    mesh = plsc.VectorSubcoreMesh(
        core_axis_name="c", subcore_axis_name="s", num_cores=nc, num_subcores=ns
    )

    @functools.partial(
        pl.kernel,
        out_type=jax.ShapeDtypeStruct((b, t, d), jnp.float32),
        mesh=mesh,
        scratch_types=[
            pltpu.VMEM((2, chunk, d), jnp.float32),
            pltpu.VMEM((2, b, chunk, d), jnp.float32),
            pltpu.SemaphoreType.DMA((2,)),
            pltpu.SemaphoreType.DMA((2,)),
        ],
    )
    def sc_fn(x_hbm, pe_hbm, out_hbm, pe2, xb, in_sems, out_sems):
        wid = lax.axis_index("s") * nc + lax.axis_index("c")
        t0 = wid * rows_per_w

        def issue_in(s, p):
            row0 = t0 + s * chunk
            pltpu.async_copy(pe_hbm.at[pl.ds(row0, chunk)], pe2.at[p], in_sems.at[p])
            for bb in range(b):
                pltpu.async_copy(
                    x_hbm.at[bb, pl.ds(row0, chunk)], xb.at[p, bb], in_sems.at[p]
                )

        def wait_in(p):
            pltpu.make_async_copy(
                pe_hbm.at[pl.ds(0, chunk)], pe2.at[p], in_sems.at[p]
            ).wait()
            for bb in range(b):
                pltpu.make_async_copy(
                    x_hbm.at[0, pl.ds(0, chunk)], xb.at[p, bb], in_sems.at[p]
                ).wait()

        def issue_out(s, p):
            row0 = t0 + s * chunk
            for bb in range(b):
                pltpu.async_copy(
                    xb.at[p, bb], out_hbm.at[bb, pl.ds(row0, chunk)], out_sems.at[p]
                )

        def wait_out(p):
            for bb in range(b):
                pltpu.make_async_copy(
                    xb.at[p, bb], out_hbm.at[0, pl.ds(0, chunk)], out_sems.at[p]
                ).wait()

        issue_in(0, 0)

        def set_body(h, carry):
            for p in range(2):
                s = h * 2 + p
                wait_in(p)

                @pl.when(s + 1 < n_sets)
                def _prefetch():
                    @pl.when(s >= 1)
                    def _drain():
                        wait_out(1 - p)

                    issue_in(s + 1, 1 - p)

                @plsc.parallel_loop(0, chunk, 1, unroll=2)
                def row_body(r):
                    grp = 4
                    for g in range(0, d // nl, grp):
                        sls = [pl.ds((g + u) * nl, nl) for u in range(grp)]
                        vals = [pe2[p, r, sls[u]] for u in range(grp)]
                        for bb in range(b):
                            for u in range(grp):
                                plsc.addupdate(xb.at[p, bb, r, sls[u]], vals[u])

                issue_out(s, p)

            return carry

        lax.fori_loop(0, n_sets // 2, set_body, 0)
        for p in range(2):
            wait_out(p)

    return sc_fn(x, pe)
